# baseline (speedup 1.0000x reference)
import jax
import jax.numpy as jnp
from jax import lax
from jax.experimental import pallas as pl
from jax.experimental.pallas import tpu as pltpu
from jax.experimental.pallas import tpu_sc as plsc

F32 = jnp.float32
NSUB = 16
NCORE = 2
CHUNK = 128


def _make_segsum(NP, KCH, W):
    RT = NP // NSUB
    mesh = plsc.VectorSubcoreMesh(core_axis_name="c", subcore_axis_name="s")

    assert KCH % 2 == 0

    def body(table, gidx, didx, zrow,
             agg_out, gidx_v, didx_v, rows_a, rows_b, agg_sh, sem_a, sem_b):
        c = lax.axis_index("c")
        s = lax.axis_index("s")
        r0 = s * RT
        pltpu.sync_copy(zrow, agg_sh.at[pl.ds(r0, RT)])
        pltpu.sync_copy(gidx.at[c, s], gidx_v)
        pltpu.sync_copy(didx.at[s], didx_v)
        plsc.subcore_barrier()

        pltpu.async_copy(table.at[gidx_v.at[0]], rows_a, sem_a)

        def chunk2(jj, carry):
            j = 2 * jj
            pltpu.make_async_copy(table.at[gidx_v.at[j]], rows_a, sem_a).wait()
            pltpu.async_copy(table.at[gidx_v.at[j + 1]], rows_b, sem_b)
            pltpu.sync_copy(rows_a, agg_sh.at[didx_v.at[j]], add=True)

            @pl.when(jj < KCH // 2 - 1)
            def _():
                pltpu.async_copy(table.at[gidx_v.at[j + 2]], rows_a, sem_a)

            pltpu.make_async_copy(
                table.at[gidx_v.at[j + 1]], rows_b, sem_b).wait()
            pltpu.sync_copy(rows_b, agg_sh.at[didx_v.at[j + 1]], add=True)
            return carry

        lax.fori_loop(0, KCH // 2, chunk2, 0)
        plsc.subcore_barrier()
        pltpu.sync_copy(agg_sh.at[pl.ds(r0, RT)], agg_out.at[c, pl.ds(r0, RT)])

    return pl.kernel(
        body,
        out_type=[jax.ShapeDtypeStruct((NCORE, NP, W), F32)],
        mesh=mesh,
        scratch_types=[
            pltpu.VMEM((KCH, CHUNK), jnp.int32),
            pltpu.VMEM((KCH, CHUNK), jnp.int32),
            pltpu.VMEM((CHUNK, W), F32),
            pltpu.VMEM((CHUNK, W), F32),
            pltpu.VMEM_SHARED((NP, W), F32),
            pltpu.SemaphoreType.DMA,
            pltpu.SemaphoreType.DMA,
        ])


def _make_segsum_const(NP, KCH, W):
    RT = NP // NSUB
    mesh = plsc.VectorSubcoreMesh(core_axis_name="c", subcore_axis_name="s")

    def body(const_rows, didx, zrow, agg_out, didx_v, rows_v, agg_sh):
        c = lax.axis_index("c")
        s = lax.axis_index("s")
        r0 = s * RT
        pltpu.sync_copy(zrow, agg_sh.at[pl.ds(r0, RT)])
        pltpu.sync_copy(didx.at[s], didx_v)
        pltpu.sync_copy(const_rows, rows_v)
        plsc.subcore_barrier()

        def chunk(j, carry):
            pltpu.sync_copy(rows_v, agg_sh.at[didx_v.at[j]], add=True)
            return carry

        lax.fori_loop(0, KCH, chunk, 0)
        plsc.subcore_barrier()
        pltpu.sync_copy(agg_sh.at[pl.ds(r0, RT)], agg_out.at[c, pl.ds(r0, RT)])

    return pl.kernel(
        body,
        out_type=[jax.ShapeDtypeStruct((NCORE, NP, W), F32)],
        mesh=mesh,
        scratch_types=[
            pltpu.VMEM((KCH, CHUNK), jnp.int32),
            pltpu.VMEM((CHUNK, W), F32),
            pltpu.VMEM_SHARED((NP, W), F32),
        ])


def _elu(v):
    return jnp.where(v > 0, v, jnp.exp(jnp.minimum(v, 0.0)) - 1.0)


def _mm_body(x_ref, w_ref, a_ref, b_ref):
    acc = jnp.dot(x_ref[...], w_ref[...], preferred_element_type=F32)
    h = a_ref.shape[1]
    a_ref[...] = acc[:, :h]
    b_ref[...] = acc[:, h:]


def _layer_body(agg_ref, cnt_ref, xr_ref, bl_ref, w_ref, hl_ref, hr_ref):
    cnt = jnp.maximum(cnt_ref[0, :, 0:1], 1.0)
    agg = jnp.concatenate([agg_ref[0], agg_ref[1]], axis=1)
    o = agg / cnt + bl_ref[...] + xr_ref[...]
    nrm = jnp.sqrt(jnp.sum(o * o, axis=-1, keepdims=True))
    h = _elu(o / jnp.maximum(nrm, 1e-12))
    hcat = jnp.dot(h, w_ref[...], preferred_element_type=F32)
    hw = hl_ref.shape[1]
    hl_ref[...] = hcat[:, :hw]
    hr_ref[...] = hcat[:, hw:]


def _head_body(agg_ref, cnt_ref, hr_ref, bl_ref, w1_ref, b1_ref,
               w2_ref, b2_ref, w3_ref, b3_ref, out_ref):
    cnt = jnp.maximum(cnt_ref[0, :, 0:1], 1.0)
    agg = jnp.concatenate([agg_ref[0], agg_ref[1]], axis=1)
    o = agg / cnt + bl_ref[...] + hr_ref[...]
    nrm = jnp.sqrt(jnp.sum(o * o, axis=-1, keepdims=True))
    h = _elu(o / jnp.maximum(nrm, 1e-12))
    h = _elu(jnp.dot(h, w1_ref[...], preferred_element_type=F32) + b1_ref[...])
    h = _elu(jnp.dot(h, w2_ref[...], preferred_element_type=F32) + b2_ref[...])
    out_ref[...] = (jnp.dot(h, w3_ref[...], preferred_element_type=F32)
                    + b3_ref[...])


def kernel(x, edges, Wl1, bl1, Wr1, Wl2, bl2, Wr2, W1, b1, W2, b2, W3, b3):
    N, Fin = x.shape
    H = Wl1.shape[0]
    Out = W3.shape[0]
    E = edges.shape[1]
    HH = H // NCORE

    NP = ((N + 1 + CHUNK - 1) // CHUNK) * CHUNK
    ET = ((E + 2 * NSUB * CHUNK - 1) // (2 * NSUB * CHUNK)) * 2 * CHUNK
    KCH = ET // CHUNK
    EP = NSUB * ET

    src = jnp.concatenate([edges[0], jnp.zeros((EP - E,), jnp.int32)])
    dst = jnp.concatenate([edges[1], jnp.full((EP - E,), N, jnp.int32)])
    gsrc = src * NCORE
    gidx = jnp.stack([gsrc + c for c in range(NCORE)]).reshape(
        NCORE, NSUB, KCH, CHUNK)
    didx = dst.reshape(NSUB, KCH, CHUNK)
    zrow = jnp.zeros((NP // NSUB, HH), F32)
    ones_rows = jnp.ones((CHUNK, HH), F32)

    Wcat1 = jnp.concatenate([Wl1.T, Wr1.T], axis=1)
    Wcat2 = jnp.concatenate([Wl2.T, Wr2.T], axis=1)
    OutP = ((Out + 127) // 128) * 128
    W3p = jnp.pad(W3.T, ((0, 0), (0, OutP - Out)))
    b3p = jnp.pad(b3, (0, OutP - Out)).reshape(1, OutP)

    RB = 400
    G = N // RB

    xl, xr = pl.pallas_call(
        _mm_body,
        grid=(G,),
        in_specs=[
            pl.BlockSpec((RB, Fin), lambda i: (i, 0)),
            pl.BlockSpec((Fin, 2 * H), lambda i: (0, 0)),
        ],
        out_specs=[
            pl.BlockSpec((RB, H), lambda i: (i, 0)),
            pl.BlockSpec((RB, H), lambda i: (i, 0)),
        ],
        out_shape=[jax.ShapeDtypeStruct((N, H), F32)] * 2,
    )(x, Wcat1)

    segsum = _make_segsum(NP, KCH, HH)

    (agg1,) = segsum(xl.reshape(NCORE * N, HH), gidx, didx, zrow)
    (cnt,) = _make_segsum_const(NP, KCH, HH)(ones_rows, didx, zrow)

    hl, hr = pl.pallas_call(
        _layer_body,
        grid=(G,),
        in_specs=[
            pl.BlockSpec((NCORE, RB, HH), lambda i: (0, i, 0)),
            pl.BlockSpec((1, RB, HH), lambda i: (0, i, 0)),
            pl.BlockSpec((RB, H), lambda i: (i, 0)),
            pl.BlockSpec((1, H), lambda i: (0, 0)),
            pl.BlockSpec((H, 2 * H), lambda i: (0, 0)),
        ],
        out_specs=[
            pl.BlockSpec((RB, H), lambda i: (i, 0)),
            pl.BlockSpec((RB, H), lambda i: (i, 0)),
        ],
        out_shape=[jax.ShapeDtypeStruct((N, H), F32)] * 2,
    )(agg1, cnt, xr, bl1.reshape(1, H), Wcat2)

    (agg2,) = segsum(hl.reshape(NCORE * N, HH), gidx, didx, zrow)

    out = pl.pallas_call(
        _head_body,
        grid=(G,),
        in_specs=[
            pl.BlockSpec((NCORE, RB, HH), lambda i: (0, i, 0)),
            pl.BlockSpec((1, RB, HH), lambda i: (0, i, 0)),
            pl.BlockSpec((RB, H), lambda i: (i, 0)),
            pl.BlockSpec((1, H), lambda i: (0, 0)),
            pl.BlockSpec((H, H), lambda i: (0, 0)),
            pl.BlockSpec((1, H), lambda i: (0, 0)),
            pl.BlockSpec((H, H), lambda i: (0, 0)),
            pl.BlockSpec((1, H), lambda i: (0, 0)),
            pl.BlockSpec((H, OutP), lambda i: (0, 0)),
            pl.BlockSpec((1, OutP), lambda i: (0, 0)),
        ],
        out_specs=pl.BlockSpec((RB, OutP), lambda i: (i, 0)),
        out_shape=jax.ShapeDtypeStruct((N, OutP), F32),
    )(agg2, cnt, hr, bl2.reshape(1, H), W1.T, b1.reshape(1, H),
      W2.T, b2.reshape(1, H), W3p, b3p)

    return out[:, :Out]


# speedup vs baseline: 1.4303x; 1.4303x over previous
import jax
import jax.numpy as jnp
from jax import lax
from jax.experimental import pallas as pl
from jax.experimental.pallas import tpu as pltpu
from jax.experimental.pallas import tpu_sc as plsc

F32 = jnp.float32
NSUB = 16
NCORE = 2
CHUNK = 128


def _make_segsum(NP, KCH, W):
    RT = NP // NSUB
    mesh = plsc.VectorSubcoreMesh(core_axis_name="c", subcore_axis_name="s")

    def body(table, gidx, didx, zrow,
             agg_out, gidx_v, didx_v, rows_v, agg_sh, sem):
        c = lax.axis_index("c")
        s = lax.axis_index("s")
        r0 = s * RT
        pltpu.sync_copy(zrow, agg_sh.at[pl.ds(r0, RT)])
        pltpu.sync_copy(gidx.at[c, s], gidx_v)
        pltpu.sync_copy(didx.at[s], didx_v)
        plsc.subcore_barrier()

        def chunk(j, carry):
            pltpu.async_copy(table.at[gidx_v.at[j]], rows_v, sem).wait()
            pltpu.sync_copy(rows_v, agg_sh.at[didx_v.at[j]], add=True)
            return carry

        lax.fori_loop(0, KCH, chunk, 0)
        plsc.subcore_barrier()
        pltpu.sync_copy(agg_sh.at[pl.ds(r0, RT)], agg_out.at[c, pl.ds(r0, RT)])

    return pl.kernel(
        body,
        out_type=[jax.ShapeDtypeStruct((NCORE, NP, W), F32)],
        mesh=mesh,
        scratch_types=[
            pltpu.VMEM((KCH, CHUNK), jnp.int32),
            pltpu.VMEM((KCH, CHUNK), jnp.int32),
            pltpu.VMEM((CHUNK, W), F32),
            pltpu.VMEM_SHARED((NP, W), F32),
            pltpu.SemaphoreType.DMA,
        ])


def _make_segsum_const(NP, KCH, W):
    RT = NP // NSUB
    mesh = plsc.VectorSubcoreMesh(core_axis_name="c", subcore_axis_name="s")

    def body(const_rows, didx, zrow, agg_out, didx_v, rows_v, agg_sh):
        c = lax.axis_index("c")
        s = lax.axis_index("s")
        r0 = s * RT
        pltpu.sync_copy(zrow, agg_sh.at[pl.ds(r0, RT)])
        pltpu.sync_copy(didx.at[s], didx_v)
        pltpu.sync_copy(const_rows, rows_v)
        plsc.subcore_barrier()

        def chunk(j, carry):
            pltpu.sync_copy(rows_v, agg_sh.at[didx_v.at[j]], add=True)
            return carry

        lax.fori_loop(0, KCH, chunk, 0)
        plsc.subcore_barrier()
        pltpu.sync_copy(agg_sh.at[pl.ds(r0, RT)], agg_out.at[c, pl.ds(r0, RT)])

    return pl.kernel(
        body,
        out_type=[jax.ShapeDtypeStruct((NCORE, NP, W), F32)],
        mesh=mesh,
        scratch_types=[
            pltpu.VMEM((KCH, CHUNK), jnp.int32),
            pltpu.VMEM((CHUNK, W), F32),
            pltpu.VMEM_SHARED((NP, W), F32),
        ])


def _elu(v):
    return jnp.where(v > 0, v, jnp.exp(jnp.minimum(v, 0.0)) - 1.0)


def _mm_body(x_ref, w_ref, o_ref):
    o_ref[...] = jnp.dot(x_ref[...], w_ref[...], preferred_element_type=F32)


def _layer_a_body(agg_ref, cnt_ref, xr_ref, bl_ref, w_ref, h_ref, hl_ref):
    cnt = jnp.maximum(cnt_ref[0, :, 0:1], 1.0)
    agg = jnp.concatenate([agg_ref[0], agg_ref[1]], axis=1)
    o = agg / cnt + bl_ref[...] + xr_ref[...]
    nrm = jnp.sqrt(jnp.sum(o * o, axis=-1, keepdims=True))
    h = _elu(o / jnp.maximum(nrm, 1e-12))
    h_ref[...] = h
    hl_ref[...] = jnp.dot(h, w_ref[...], preferred_element_type=F32)


def _head_body(agg_ref, cnt_ref, hr_ref, bl_ref, w1_ref, b1_ref,
               w2_ref, b2_ref, w3_ref, b3_ref, out_ref):
    cnt = jnp.maximum(cnt_ref[0, :, 0:1], 1.0)
    agg = jnp.concatenate([agg_ref[0], agg_ref[1]], axis=1)
    o = agg / cnt + bl_ref[...] + hr_ref[...]
    nrm = jnp.sqrt(jnp.sum(o * o, axis=-1, keepdims=True))
    h = _elu(o / jnp.maximum(nrm, 1e-12))
    h = _elu(jnp.dot(h, w1_ref[...], preferred_element_type=F32) + b1_ref[...])
    h = _elu(jnp.dot(h, w2_ref[...], preferred_element_type=F32) + b2_ref[...])
    out_ref[...] = (jnp.dot(h, w3_ref[...], preferred_element_type=F32)
                    + b3_ref[...])


def kernel(x, edges, Wl1, bl1, Wr1, Wl2, bl2, Wr2, W1, b1, W2, b2, W3, b3):
    N, Fin = x.shape
    H = Wl1.shape[0]
    Out = W3.shape[0]
    E = edges.shape[1]
    HH = H // NCORE

    NP = ((N + 1 + CHUNK - 1) // CHUNK) * CHUNK
    ET = ((E + NSUB * CHUNK - 1) // (NSUB * CHUNK)) * CHUNK
    KCH = ET // CHUNK
    EP = NSUB * ET

    src = jnp.concatenate([edges[0], jnp.zeros((EP - E,), jnp.int32)])
    dst = jnp.concatenate([edges[1], jnp.full((EP - E,), N, jnp.int32)])
    gsrc = src * NCORE
    gidx = jnp.stack([gsrc + c for c in range(NCORE)]).reshape(
        NCORE, NSUB, KCH, CHUNK)
    didx = dst.reshape(NSUB, KCH, CHUNK)
    zrow = jnp.zeros((NP // NSUB, HH), F32)
    ones_rows = jnp.ones((CHUNK, HH), F32)

    OutP = ((Out + 127) // 128) * 128
    W3p = jnp.pad(W3.T, ((0, 0), (0, OutP - Out)))
    b3p = jnp.pad(b3, (0, OutP - Out)).reshape(1, OutP)

    RB = 400
    G = N // RB

    def mm(a, w):
        m, k = a.shape
        n = w.shape[1]
        return pl.pallas_call(
            _mm_body,
            grid=(m // RB,),
            in_specs=[
                pl.BlockSpec((RB, k), lambda i: (i, 0)),
                pl.BlockSpec((k, n), lambda i: (0, 0)),
            ],
            out_specs=pl.BlockSpec((RB, n), lambda i: (i, 0)),
            out_shape=jax.ShapeDtypeStruct((m, n), F32),
        )(a, w)

    segsum = _make_segsum(NP, KCH, HH)

    (cnt,) = _make_segsum_const(NP, KCH, HH)(ones_rows, didx, zrow)
    xl = mm(x, Wl1.T)
    (agg1,) = segsum(xl.reshape(NCORE * N, HH), gidx, didx, zrow)
    xr = mm(x, Wr1.T)

    h1, hl = pl.pallas_call(
        _layer_a_body,
        grid=(G,),
        in_specs=[
            pl.BlockSpec((NCORE, RB, HH), lambda i: (0, i, 0)),
            pl.BlockSpec((1, RB, HH), lambda i: (0, i, 0)),
            pl.BlockSpec((RB, H), lambda i: (i, 0)),
            pl.BlockSpec((1, H), lambda i: (0, 0)),
            pl.BlockSpec((H, H), lambda i: (0, 0)),
        ],
        out_specs=[
            pl.BlockSpec((RB, H), lambda i: (i, 0)),
            pl.BlockSpec((RB, H), lambda i: (i, 0)),
        ],
        out_shape=[jax.ShapeDtypeStruct((N, H), F32)] * 2,
    )(agg1, cnt, xr, bl1.reshape(1, H), Wl2.T)

    (agg2,) = segsum(hl.reshape(NCORE * N, HH), gidx, didx, zrow)
    hr = mm(h1, Wr2.T)

    out = pl.pallas_call(
        _head_body,
        grid=(G,),
        in_specs=[
            pl.BlockSpec((NCORE, RB, HH), lambda i: (0, i, 0)),
            pl.BlockSpec((1, RB, HH), lambda i: (0, i, 0)),
            pl.BlockSpec((RB, H), lambda i: (i, 0)),
            pl.BlockSpec((1, H), lambda i: (0, 0)),
            pl.BlockSpec((H, H), lambda i: (0, 0)),
            pl.BlockSpec((1, H), lambda i: (0, 0)),
            pl.BlockSpec((H, H), lambda i: (0, 0)),
            pl.BlockSpec((1, H), lambda i: (0, 0)),
            pl.BlockSpec((H, OutP), lambda i: (0, 0)),
            pl.BlockSpec((1, OutP), lambda i: (0, 0)),
        ],
        out_specs=pl.BlockSpec((RB, OutP), lambda i: (i, 0)),
        out_shape=jax.ShapeDtypeStruct((N, OutP), F32),
    )(agg2, cnt, hr, bl2.reshape(1, H), W1.T, b1.reshape(1, H),
      W2.T, b2.reshape(1, H), W3p, b3p)

    return out[:, :Out]


# speedup vs baseline: 1.4664x; 1.0252x over previous
import jax
import jax.numpy as jnp
from jax import lax
from jax.experimental import pallas as pl
from jax.experimental.pallas import tpu as pltpu
from jax.experimental.pallas import tpu_sc as plsc

F32 = jnp.float32
NSUB = 16
NCORE = 2
CHUNK = 128


def _make_segsum(NP, KCH, W):
    RT = NP // NSUB
    mesh = plsc.VectorSubcoreMesh(core_axis_name="c", subcore_axis_name="s")

    def body(table, gidx, didx, zrow,
             agg_out, gidx_v, didx_v, rows_v, agg_sh, sem):
        c = lax.axis_index("c")
        s = lax.axis_index("s")
        r0 = s * RT
        pltpu.sync_copy(zrow, agg_sh.at[pl.ds(r0, RT)])
        pltpu.sync_copy(gidx.at[c, s], gidx_v)
        pltpu.sync_copy(didx.at[s], didx_v)
        plsc.subcore_barrier()

        def chunk(j, carry):
            pltpu.async_copy(table.at[gidx_v.at[j]], rows_v, sem).wait()
            pltpu.sync_copy(rows_v, agg_sh.at[didx_v.at[j]], add=True)
            return carry

        lax.fori_loop(0, KCH, chunk, 0)
        plsc.subcore_barrier()
        pltpu.sync_copy(agg_sh.at[pl.ds(r0, RT)], agg_out.at[c, pl.ds(r0, RT)])

    return pl.kernel(
        body,
        out_type=[jax.ShapeDtypeStruct((NCORE, NP, W), F32)],
        mesh=mesh,
        scratch_types=[
            pltpu.VMEM((KCH, CHUNK), jnp.int32),
            pltpu.VMEM((KCH, CHUNK), jnp.int32),
            pltpu.VMEM((CHUNK, W), F32),
            pltpu.VMEM_SHARED((NP, W), F32),
            pltpu.SemaphoreType.DMA,
        ])


def _make_segsum_const(NP, KCH, W):
    RT = NP // NSUB
    mesh = plsc.VectorSubcoreMesh(core_axis_name="c", subcore_axis_name="s")

    def body(const_rows, didx, zrow, agg_out, didx_v, rows_v, agg_sh):
        c = lax.axis_index("c")
        s = lax.axis_index("s")
        r0 = s * RT
        pltpu.sync_copy(zrow, agg_sh.at[pl.ds(r0, RT)])
        pltpu.sync_copy(didx.at[s], didx_v)
        pltpu.sync_copy(const_rows, rows_v)
        plsc.subcore_barrier()

        def chunk(j, carry):
            pltpu.sync_copy(rows_v, agg_sh.at[didx_v.at[j]], add=True)
            return carry

        lax.fori_loop(0, KCH, chunk, 0)
        plsc.subcore_barrier()
        pltpu.sync_copy(agg_sh.at[pl.ds(r0, RT)], agg_out.at[c, pl.ds(r0, RT)])

    return pl.kernel(
        body,
        out_type=[jax.ShapeDtypeStruct((NCORE, NP, W), F32)],
        mesh=mesh,
        scratch_types=[
            pltpu.VMEM((KCH, CHUNK), jnp.int32),
            pltpu.VMEM((CHUNK, W), F32),
            pltpu.VMEM_SHARED((NP, W), F32),
        ])


def _elu(v):
    return jnp.where(v > 0, v, jnp.exp(jnp.minimum(v, 0.0)) - 1.0)


def _mm_body(x_ref, w_ref, o_ref):
    o_ref[...] = jnp.dot(x_ref[...], w_ref[...], preferred_element_type=F32)


def _mm2_body(x_ref, w_ref, a_ref, b_ref):
    acc = jnp.dot(x_ref[...].astype(jnp.bfloat16), w_ref[...],
                  preferred_element_type=F32)
    h = a_ref.shape[1]
    a_ref[...] = acc[:, :h]
    b_ref[...] = acc[:, h:]


def _layer_a_body(agg_ref, cnt_ref, xr_ref, bl_ref, w_ref, h_ref, hl_ref):
    cnt = jnp.maximum(cnt_ref[0, :, 0:1], 1.0)
    agg = jnp.concatenate([agg_ref[0], agg_ref[1]], axis=1)
    o = agg / cnt + bl_ref[...] + xr_ref[...]
    nrm = jnp.sqrt(jnp.sum(o * o, axis=-1, keepdims=True))
    h = _elu(o / jnp.maximum(nrm, 1e-12))
    h_ref[...] = h
    hl_ref[...] = jnp.dot(h, w_ref[...], preferred_element_type=F32)


def _head_body(agg_ref, cnt_ref, hr_ref, bl_ref, w1_ref, b1_ref,
               w2_ref, b2_ref, w3_ref, b3_ref, out_ref):
    cnt = jnp.maximum(cnt_ref[0, :, 0:1], 1.0)
    agg = jnp.concatenate([agg_ref[0], agg_ref[1]], axis=1)
    o = agg / cnt + bl_ref[...] + hr_ref[...]
    nrm = jnp.sqrt(jnp.sum(o * o, axis=-1, keepdims=True))
    h = _elu(o / jnp.maximum(nrm, 1e-12))
    h = _elu(jnp.dot(h, w1_ref[...], preferred_element_type=F32) + b1_ref[...])
    h = _elu(jnp.dot(h, w2_ref[...], preferred_element_type=F32) + b2_ref[...])
    out_ref[...] = (jnp.dot(h, w3_ref[...], preferred_element_type=F32)
                    + b3_ref[...])


def kernel(x, edges, Wl1, bl1, Wr1, Wl2, bl2, Wr2, W1, b1, W2, b2, W3, b3):
    N, Fin = x.shape
    H = Wl1.shape[0]
    Out = W3.shape[0]
    E = edges.shape[1]
    HH = H // NCORE

    NP = ((N + 1 + CHUNK - 1) // CHUNK) * CHUNK
    ET = ((E + NSUB * CHUNK - 1) // (NSUB * CHUNK)) * CHUNK
    KCH = ET // CHUNK
    EP = NSUB * ET

    src = jnp.concatenate([edges[0], jnp.zeros((EP - E,), jnp.int32)])
    dst = jnp.concatenate([edges[1], jnp.full((EP - E,), N, jnp.int32)])
    gsrc = src * NCORE
    gidx = jnp.stack([gsrc + c for c in range(NCORE)]).reshape(
        NCORE, NSUB, KCH, CHUNK)
    didx = dst.reshape(NSUB, KCH, CHUNK)
    zrow = jnp.zeros((NP // NSUB, HH), F32)
    ones_rows = jnp.ones((CHUNK, HH), F32)

    OutP = ((Out + 127) // 128) * 128
    W3p = jnp.pad(W3.T, ((0, 0), (0, OutP - Out)))
    b3p = jnp.pad(b3, (0, OutP - Out)).reshape(1, OutP)

    RB = 400
    G = N // RB

    def mm(a, w):
        m, k = a.shape
        n = w.shape[1]
        return pl.pallas_call(
            _mm_body,
            grid=(m // RB,),
            in_specs=[
                pl.BlockSpec((RB, k), lambda i: (i, 0)),
                pl.BlockSpec((k, n), lambda i: (0, 0)),
            ],
            out_specs=pl.BlockSpec((RB, n), lambda i: (i, 0)),
            out_shape=jax.ShapeDtypeStruct((m, n), F32),
        )(a, w)

    segsum = _make_segsum(NP, KCH, HH)

    (cnt,) = _make_segsum_const(NP, KCH, HH)(ones_rows, didx, zrow)
    Wcat1 = jnp.concatenate([Wl1.T, Wr1.T], axis=1).astype(jnp.bfloat16)
    xl, xr = pl.pallas_call(
        _mm2_body,
        grid=(G,),
        in_specs=[
            pl.BlockSpec((RB, Fin), lambda i: (i, 0)),
            pl.BlockSpec((Fin, 2 * H), lambda i: (0, 0)),
        ],
        out_specs=[
            pl.BlockSpec((RB, H), lambda i: (i, 0)),
            pl.BlockSpec((RB, H), lambda i: (i, 0)),
        ],
        out_shape=[jax.ShapeDtypeStruct((N, H), F32)] * 2,
    )(x, Wcat1)
    (agg1,) = segsum(xl.reshape(NCORE * N, HH), gidx, didx, zrow)

    h1, hl = pl.pallas_call(
        _layer_a_body,
        grid=(G,),
        in_specs=[
            pl.BlockSpec((NCORE, RB, HH), lambda i: (0, i, 0)),
            pl.BlockSpec((1, RB, HH), lambda i: (0, i, 0)),
            pl.BlockSpec((RB, H), lambda i: (i, 0)),
            pl.BlockSpec((1, H), lambda i: (0, 0)),
            pl.BlockSpec((H, H), lambda i: (0, 0)),
        ],
        out_specs=[
            pl.BlockSpec((RB, H), lambda i: (i, 0)),
            pl.BlockSpec((RB, H), lambda i: (i, 0)),
        ],
        out_shape=[jax.ShapeDtypeStruct((N, H), F32)] * 2,
    )(agg1, cnt, xr, bl1.reshape(1, H), Wl2.T)

    (agg2,) = segsum(hl.reshape(NCORE * N, HH), gidx, didx, zrow)
    hr = mm(h1, Wr2.T)

    out = pl.pallas_call(
        _head_body,
        grid=(G,),
        in_specs=[
            pl.BlockSpec((NCORE, RB, HH), lambda i: (0, i, 0)),
            pl.BlockSpec((1, RB, HH), lambda i: (0, i, 0)),
            pl.BlockSpec((RB, H), lambda i: (i, 0)),
            pl.BlockSpec((1, H), lambda i: (0, 0)),
            pl.BlockSpec((H, H), lambda i: (0, 0)),
            pl.BlockSpec((1, H), lambda i: (0, 0)),
            pl.BlockSpec((H, H), lambda i: (0, 0)),
            pl.BlockSpec((1, H), lambda i: (0, 0)),
            pl.BlockSpec((H, OutP), lambda i: (0, 0)),
            pl.BlockSpec((1, OutP), lambda i: (0, 0)),
        ],
        out_specs=pl.BlockSpec((RB, OutP), lambda i: (i, 0)),
        out_shape=jax.ShapeDtypeStruct((N, OutP), F32),
    )(agg2, cnt, hr, bl2.reshape(1, H), W1.T, b1.reshape(1, H),
      W2.T, b2.reshape(1, H), W3p, b3p)

    return out[:, :Out]


# speedup vs baseline: 1.4671x; 1.0005x over previous
import jax
import jax.numpy as jnp
from jax import lax
from jax.experimental import pallas as pl
from jax.experimental.pallas import tpu as pltpu
from jax.experimental.pallas import tpu_sc as plsc

F32 = jnp.float32
NSUB = 16
NCORE = 2
CHUNK = 128


def _make_segsum(NP, KCH, W):
    RT = NP // NSUB
    mesh = plsc.VectorSubcoreMesh(core_axis_name="c", subcore_axis_name="s")

    def body(table, gidx, didx, zrow,
             agg_out, gidx_v, didx_v, rows_v, agg_sh, sem):
        c = lax.axis_index("c")
        s = lax.axis_index("s")
        r0 = s * RT
        pltpu.sync_copy(zrow, agg_sh.at[pl.ds(r0, RT)])
        pltpu.sync_copy(gidx.at[c, s], gidx_v)
        pltpu.sync_copy(didx.at[s], didx_v)
        plsc.subcore_barrier()

        def chunk(j, carry):
            pltpu.async_copy(table.at[gidx_v.at[j]], rows_v, sem).wait()
            pltpu.sync_copy(rows_v, agg_sh.at[didx_v.at[j]], add=True)
            return carry

        lax.fori_loop(0, KCH, chunk, 0)
        plsc.subcore_barrier()
        pltpu.sync_copy(agg_sh.at[pl.ds(r0, RT)], agg_out.at[c, pl.ds(r0, RT)])

    return pl.kernel(
        body,
        out_type=[jax.ShapeDtypeStruct((NCORE, NP, W), F32)],
        mesh=mesh,
        scratch_types=[
            pltpu.VMEM((KCH, CHUNK), jnp.int32),
            pltpu.VMEM((KCH, CHUNK), jnp.int32),
            pltpu.VMEM((CHUNK, W), F32),
            pltpu.VMEM_SHARED((NP, W), F32),
            pltpu.SemaphoreType.DMA,
        ])


def _make_segsum_const(NP, KCH, W):
    RT = NP // NSUB
    mesh = plsc.VectorSubcoreMesh(core_axis_name="c", subcore_axis_name="s")

    def body(const_rows, didx, zrow, agg_out, didx_v, rows_v, agg_sh):
        c = lax.axis_index("c")
        s = lax.axis_index("s")
        r0 = s * RT
        pltpu.sync_copy(zrow, agg_sh.at[pl.ds(r0, RT)])
        pltpu.sync_copy(didx.at[s], didx_v)
        pltpu.sync_copy(const_rows, rows_v)
        plsc.subcore_barrier()

        def chunk(j, carry):
            pltpu.sync_copy(rows_v, agg_sh.at[didx_v.at[j]], add=True)
            return carry

        lax.fori_loop(0, KCH, chunk, 0)
        plsc.subcore_barrier()
        pltpu.sync_copy(agg_sh.at[pl.ds(r0, RT)], agg_out.at[c, pl.ds(r0, RT)])

    return pl.kernel(
        body,
        out_type=[jax.ShapeDtypeStruct((NCORE, NP, W), F32)],
        mesh=mesh,
        scratch_types=[
            pltpu.VMEM((KCH, CHUNK), jnp.int32),
            pltpu.VMEM((CHUNK, W), F32),
            pltpu.VMEM_SHARED((NP, W), F32),
        ])


def _elu(v):
    return jnp.where(v > 0, v, jnp.exp(jnp.minimum(v, 0.0)) - 1.0)


def _mm2_body(x_ref, w_ref, a_ref, b_ref):
    acc = jnp.dot(x_ref[...], w_ref[...], preferred_element_type=F32)
    h = a_ref.shape[1]
    a_ref[...] = acc[:, :h]
    b_ref[...] = acc[:, h:]


def _layer_body(agg_ref, cnt_ref, xr_ref, bl_ref, w_ref, hl_ref, hr_ref):
    cnt = jnp.maximum(cnt_ref[0, :, 0:1], 1.0)
    agg = jnp.concatenate([agg_ref[0], agg_ref[1]], axis=1)
    o = agg / cnt + bl_ref[...] + xr_ref[...]
    nrm = jnp.sqrt(jnp.sum(o * o, axis=-1, keepdims=True))
    h = _elu(o / jnp.maximum(nrm, 1e-12))
    hcat = jnp.dot(h, w_ref[...], preferred_element_type=F32)
    hw = hl_ref.shape[1]
    hl_ref[...] = hcat[:, :hw]
    hr_ref[...] = hcat[:, hw:]


def _head_body(agg_ref, cnt_ref, hr_ref, bl_ref, w1_ref, b1_ref,
               w2_ref, b2_ref, w3_ref, b3_ref, out_ref):
    cnt = jnp.maximum(cnt_ref[0, :, 0:1], 1.0)
    agg = jnp.concatenate([agg_ref[0], agg_ref[1]], axis=1)
    o = agg / cnt + bl_ref[...] + hr_ref[...]
    nrm = jnp.sqrt(jnp.sum(o * o, axis=-1, keepdims=True))
    h = _elu(o / jnp.maximum(nrm, 1e-12))
    h = _elu(jnp.dot(h, w1_ref[...], preferred_element_type=F32) + b1_ref[...])
    h = _elu(jnp.dot(h, w2_ref[...], preferred_element_type=F32) + b2_ref[...])
    out_ref[...] = (jnp.dot(h, w3_ref[...], preferred_element_type=F32)
                    + b3_ref[...])


def kernel(x, edges, Wl1, bl1, Wr1, Wl2, bl2, Wr2, W1, b1, W2, b2, W3, b3):
    N, Fin = x.shape
    H = Wl1.shape[0]
    Out = W3.shape[0]
    E = edges.shape[1]
    HH = H // NCORE

    NP = ((N + 1 + CHUNK - 1) // CHUNK) * CHUNK
    ET = ((E + NSUB * CHUNK - 1) // (NSUB * CHUNK)) * CHUNK
    KCH = ET // CHUNK
    EP = NSUB * ET

    src = jnp.concatenate([edges[0], jnp.zeros((EP - E,), jnp.int32)])
    dst = jnp.concatenate([edges[1], jnp.full((EP - E,), N, jnp.int32)])
    gsrc = src * NCORE
    gidx = jnp.stack([gsrc + c for c in range(NCORE)]).reshape(
        NCORE, NSUB, KCH, CHUNK)
    didx = dst.reshape(NSUB, KCH, CHUNK)
    zrow = jnp.zeros((NP // NSUB, HH), F32)
    ones_rows = jnp.ones((CHUNK, HH), F32)

    OutP = ((Out + 127) // 128) * 128
    W3p = jnp.pad(W3.T, ((0, 0), (0, OutP - Out)))
    b3p = jnp.pad(b3, (0, OutP - Out)).reshape(1, OutP)

    RB = 400
    G = N // RB

    segsum = _make_segsum(NP, KCH, HH)

    (cnt,) = _make_segsum_const(NP, KCH, HH)(ones_rows, didx, zrow)
    Wcat1 = jnp.concatenate([Wl1.T, Wr1.T], axis=1)
    xl, xr = pl.pallas_call(
        _mm2_body,
        grid=(G,),
        in_specs=[
            pl.BlockSpec((RB, Fin), lambda i: (i, 0)),
            pl.BlockSpec((Fin, 2 * H), lambda i: (0, 0)),
        ],
        out_specs=[
            pl.BlockSpec((RB, H), lambda i: (i, 0)),
            pl.BlockSpec((RB, H), lambda i: (i, 0)),
        ],
        out_shape=[jax.ShapeDtypeStruct((N, H), F32)] * 2,
    )(x, Wcat1)
    (agg1,) = segsum(xl.reshape(NCORE * N, HH), gidx, didx, zrow)

    Wcat2 = jnp.concatenate([Wl2.T, Wr2.T], axis=1)
    hl, hr = pl.pallas_call(
        _layer_body,
        grid=(G,),
        in_specs=[
            pl.BlockSpec((NCORE, RB, HH), lambda i: (0, i, 0)),
            pl.BlockSpec((1, RB, HH), lambda i: (0, i, 0)),
            pl.BlockSpec((RB, H), lambda i: (i, 0)),
            pl.BlockSpec((1, H), lambda i: (0, 0)),
            pl.BlockSpec((H, 2 * H), lambda i: (0, 0)),
        ],
        out_specs=[
            pl.BlockSpec((RB, H), lambda i: (i, 0)),
            pl.BlockSpec((RB, H), lambda i: (i, 0)),
        ],
        out_shape=[jax.ShapeDtypeStruct((N, H), F32)] * 2,
    )(agg1, cnt, xr, bl1.reshape(1, H), Wcat2)

    (agg2,) = segsum(hl.reshape(NCORE * N, HH), gidx, didx, zrow)

    out = pl.pallas_call(
        _head_body,
        grid=(G,),
        in_specs=[
            pl.BlockSpec((NCORE, RB, HH), lambda i: (0, i, 0)),
            pl.BlockSpec((1, RB, HH), lambda i: (0, i, 0)),
            pl.BlockSpec((RB, H), lambda i: (i, 0)),
            pl.BlockSpec((1, H), lambda i: (0, 0)),
            pl.BlockSpec((H, H), lambda i: (0, 0)),
            pl.BlockSpec((1, H), lambda i: (0, 0)),
            pl.BlockSpec((H, H), lambda i: (0, 0)),
            pl.BlockSpec((1, H), lambda i: (0, 0)),
            pl.BlockSpec((H, OutP), lambda i: (0, 0)),
            pl.BlockSpec((1, OutP), lambda i: (0, 0)),
        ],
        out_specs=pl.BlockSpec((RB, OutP), lambda i: (i, 0)),
        out_shape=jax.ShapeDtypeStruct((N, OutP), F32),
    )(agg2, cnt, hr, bl2.reshape(1, H), W1.T, b1.reshape(1, H),
      W2.T, b2.reshape(1, H), W3p, b3p)

    return out[:, :Out]


# speedup vs baseline: 1.5680x; 1.0688x over previous
import jax
import jax.numpy as jnp
from jax import lax
from jax.experimental import pallas as pl
from jax.experimental.pallas import tpu as pltpu
from jax.experimental.pallas import tpu_sc as plsc

F32 = jnp.float32
NSUB = 16
NCORE = 2
CHUNK = 128


def _make_segsum(NP, KCH, W):
    RT = NP // NSUB
    mesh = plsc.VectorSubcoreMesh(core_axis_name="c", subcore_axis_name="s")

    def body(table, gidx, didx, zrow,
             agg_out, gidx_v, didx_v, rows_v, agg_sh, sem):
        c = lax.axis_index("c")
        s = lax.axis_index("s")
        r0 = s * RT
        pltpu.sync_copy(zrow, agg_sh.at[pl.ds(r0, RT)])
        pltpu.sync_copy(gidx.at[c, s], gidx_v)
        pltpu.sync_copy(didx.at[s], didx_v)
        plsc.subcore_barrier()

        def chunk(j, carry):
            pltpu.async_copy(table.at[gidx_v.at[j]], rows_v, sem).wait()
            pltpu.sync_copy(rows_v, agg_sh.at[didx_v.at[j]], add=True)
            return carry

        lax.fori_loop(0, KCH, chunk, 0)
        plsc.subcore_barrier()
        pltpu.sync_copy(agg_sh.at[pl.ds(r0, RT)], agg_out.at[c, pl.ds(r0, RT)])

    return pl.kernel(
        body,
        out_type=[jax.ShapeDtypeStruct((NCORE, NP, W), F32)],
        mesh=mesh,
        scratch_types=[
            pltpu.VMEM((KCH, CHUNK), jnp.int32),
            pltpu.VMEM((KCH, CHUNK), jnp.int32),
            pltpu.VMEM((CHUNK, W), F32),
            pltpu.VMEM_SHARED((NP, W), F32),
            pltpu.SemaphoreType.DMA,
        ])


def _make_segsum_const(NP, KCH, W):
    RT = NP // NSUB
    mesh = plsc.VectorSubcoreMesh(core_axis_name="c", subcore_axis_name="s")

    def body(const_rows, didx, zrow, agg_out, didx_v, rows_v, agg_sh):
        c = lax.axis_index("c")
        s = lax.axis_index("s")
        r0 = s * RT
        pltpu.sync_copy(zrow, agg_sh.at[pl.ds(r0, RT)])
        pltpu.sync_copy(didx.at[s], didx_v)
        pltpu.sync_copy(const_rows, rows_v)
        plsc.subcore_barrier()

        def chunk(j, carry):
            pltpu.sync_copy(rows_v, agg_sh.at[didx_v.at[j]], add=True)
            return carry

        lax.fori_loop(0, KCH, chunk, 0)
        plsc.subcore_barrier()
        pltpu.sync_copy(agg_sh.at[pl.ds(r0, RT)], agg_out.at[c, pl.ds(r0, RT)])

    return pl.kernel(
        body,
        out_type=[jax.ShapeDtypeStruct((NCORE, NP, W), F32)],
        mesh=mesh,
        scratch_types=[
            pltpu.VMEM((KCH, CHUNK), jnp.int32),
            pltpu.VMEM((CHUNK, W), F32),
            pltpu.VMEM_SHARED((NP, W), F32),
        ])


def _elu(v):
    return jnp.where(v > 0, v, jnp.exp(jnp.minimum(v, 0.0)) - 1.0)


def _mm2_body(x_ref, w_ref, a_ref, b_ref):
    acc = jnp.dot(x_ref[...], w_ref[...], preferred_element_type=F32)
    h = a_ref.shape[1]
    a_ref[...] = acc[:, :h]
    b_ref[...] = acc[:, h:]


def _layer_body(agg_ref, cnt_ref, xr_ref, bl_ref, w_ref, hl_ref, hr_ref):
    cnt = jnp.maximum(cnt_ref[0, :, 0:1], 1.0)
    agg = jnp.concatenate([agg_ref[0], agg_ref[1]], axis=1)
    o = agg / cnt + bl_ref[...] + xr_ref[...]
    nrm = jnp.sqrt(jnp.sum(o * o, axis=-1, keepdims=True))
    h = _elu(o / jnp.maximum(nrm, 1e-12))
    hcat = jnp.dot(h, w_ref[...], preferred_element_type=F32)
    hw = hl_ref.shape[1]
    hl_ref[...] = hcat[:, :hw]
    hr_ref[...] = hcat[:, hw:]


def _head_body(agg_ref, cnt_ref, hr_ref, bl_ref, w1_ref, b1_ref,
               w2_ref, b2_ref, w3_ref, b3_ref, out_ref):
    cnt = jnp.maximum(cnt_ref[0, :, 0:1], 1.0)
    agg = jnp.concatenate([agg_ref[0], agg_ref[1]], axis=1)
    o = agg / cnt + bl_ref[...] + hr_ref[...]
    nrm = jnp.sqrt(jnp.sum(o * o, axis=-1, keepdims=True))
    h = _elu(o / jnp.maximum(nrm, 1e-12))
    h = _elu(jnp.dot(h, w1_ref[...], preferred_element_type=F32) + b1_ref[...])
    h = _elu(jnp.dot(h, w2_ref[...], preferred_element_type=F32) + b2_ref[...])
    out_ref[...] = (jnp.dot(h, w3_ref[...], preferred_element_type=F32)
                    + b3_ref[...])


def kernel(x, edges, Wl1, bl1, Wr1, Wl2, bl2, Wr2, W1, b1, W2, b2, W3, b3):
    N, Fin = x.shape
    H = Wl1.shape[0]
    Out = W3.shape[0]
    E = edges.shape[1]
    HH = H // NCORE

    NP = ((N + 1 + CHUNK - 1) // CHUNK) * CHUNK
    ET = ((E + NSUB * CHUNK - 1) // (NSUB * CHUNK)) * CHUNK
    KCH = ET // CHUNK
    EP = NSUB * ET

    src = jnp.concatenate([edges[0], jnp.zeros((EP - E,), jnp.int32)])
    dst = jnp.concatenate([edges[1], jnp.full((EP - E,), N, jnp.int32)])
    gsrc = src * NCORE
    gidx = jnp.stack([gsrc + c for c in range(NCORE)]).reshape(
        NCORE, NSUB, KCH, CHUNK)
    didx = dst.reshape(NSUB, KCH, CHUNK)
    zrow = jnp.zeros((NP // NSUB, HH), F32)
    ones_rows = jnp.ones((CHUNK, HH), F32)

    OutP = ((Out + 127) // 128) * 128
    W3p = jnp.pad(W3.T, ((0, 0), (0, OutP - Out)))
    b3p = jnp.pad(b3, (0, OutP - Out)).reshape(1, OutP)

    RB = 1000
    G = N // RB

    segsum = _make_segsum(NP, KCH, HH)

    (cnt,) = _make_segsum_const(NP, KCH, HH)(ones_rows, didx, zrow)
    Wcat1 = jnp.concatenate([Wl1.T, Wr1.T], axis=1)
    xl, xr = pl.pallas_call(
        _mm2_body,
        grid=(G,),
        in_specs=[
            pl.BlockSpec((RB, Fin), lambda i: (i, 0)),
            pl.BlockSpec((Fin, 2 * H), lambda i: (0, 0)),
        ],
        out_specs=[
            pl.BlockSpec((RB, H), lambda i: (i, 0)),
            pl.BlockSpec((RB, H), lambda i: (i, 0)),
        ],
        out_shape=[jax.ShapeDtypeStruct((N, H), F32)] * 2,
    )(x, Wcat1)
    (agg1,) = segsum(xl.reshape(NCORE * N, HH), gidx, didx, zrow)

    Wcat2 = jnp.concatenate([Wl2.T, Wr2.T], axis=1)
    hl, hr = pl.pallas_call(
        _layer_body,
        grid=(G,),
        in_specs=[
            pl.BlockSpec((NCORE, RB, HH), lambda i: (0, i, 0)),
            pl.BlockSpec((1, RB, HH), lambda i: (0, i, 0)),
            pl.BlockSpec((RB, H), lambda i: (i, 0)),
            pl.BlockSpec((1, H), lambda i: (0, 0)),
            pl.BlockSpec((H, 2 * H), lambda i: (0, 0)),
        ],
        out_specs=[
            pl.BlockSpec((RB, H), lambda i: (i, 0)),
            pl.BlockSpec((RB, H), lambda i: (i, 0)),
        ],
        out_shape=[jax.ShapeDtypeStruct((N, H), F32)] * 2,
    )(agg1, cnt, xr, bl1.reshape(1, H), Wcat2)

    (agg2,) = segsum(hl.reshape(NCORE * N, HH), gidx, didx, zrow)

    out = pl.pallas_call(
        _head_body,
        grid=(G,),
        in_specs=[
            pl.BlockSpec((NCORE, RB, HH), lambda i: (0, i, 0)),
            pl.BlockSpec((1, RB, HH), lambda i: (0, i, 0)),
            pl.BlockSpec((RB, H), lambda i: (i, 0)),
            pl.BlockSpec((1, H), lambda i: (0, 0)),
            pl.BlockSpec((H, H), lambda i: (0, 0)),
            pl.BlockSpec((1, H), lambda i: (0, 0)),
            pl.BlockSpec((H, H), lambda i: (0, 0)),
            pl.BlockSpec((1, H), lambda i: (0, 0)),
            pl.BlockSpec((H, OutP), lambda i: (0, 0)),
            pl.BlockSpec((1, OutP), lambda i: (0, 0)),
        ],
        out_specs=pl.BlockSpec((RB, OutP), lambda i: (i, 0)),
        out_shape=jax.ShapeDtypeStruct((N, OutP), F32),
    )(agg2, cnt, hr, bl2.reshape(1, H), W1.T, b1.reshape(1, H),
      W2.T, b2.reshape(1, H), W3p, b3p)

    return out[:, :Out]
